# baseline (speedup 1.0000x reference)
import functools
import jax
import jax.numpy as jnp
from jax import lax
from jax.experimental import pallas as pl
from jax.experimental.pallas import tpu as pltpu
from jax.experimental.pallas import tpu_sc as plsc

_N = 10000
_E = 320000
_H = 128
_NS = 16
_EPT = _E // 2 // _NS
_CHUNK = 80
_NCH = _EPT // _CHUNK
_RPT = 624

_NBUF = 4
_IGEN = 8
_UNROLL = 8
_NFULL = _NCH // _UNROLL * _UNROLL


def _segsum_body(h_hbm, z_hbm, src_hbm, dst_hbm, out_hbm, *scr):
  rows = scr[0:4]
  sidx = scr[4:12]
  didx = scr[12:20]
  gsem = scr[20:24]
  ssem = scr[24:28]
  isem_s = scr[28:36]
  isem_d = scr[36:44]
  accum = scr[44]
  c = lax.axis_index("c")
  s = lax.axis_index("s")
  r0 = s * _RPT
  tail = _NS * _RPT
  @pl.when(c == 0)
  def _():
    pltpu.sync_copy(h_hbm.at[pl.ds(r0, _RPT)], accum.at[pl.ds(r0, _RPT)])

    @pl.when(s == _NS - 1)
    def _():
      pltpu.sync_copy(h_hbm.at[pl.ds(tail, _N - tail)],
                      accum.at[pl.ds(tail, _N - tail)])

  @pl.when(c == 1)
  def _():
    pltpu.sync_copy(z_hbm.at[pl.ds(r0, _RPT)], accum.at[pl.ds(r0, _RPT)])

    @pl.when(s == _NS - 1)
    def _():
      pltpu.sync_copy(z_hbm.at[pl.ds(tail, _N - tail)],
                      accum.at[pl.ds(tail, _N - tail)])

  plsc.subcore_barrier()

  ebase = (c * _NS + s) * _EPT

  def idx_src(i, g):
    return pltpu.make_async_copy(src_hbm.at[pl.ds(ebase + i * _CHUNK, _CHUNK)],
                                 sidx[g], isem_s[g])

  def idx_dst(i, g):
    return pltpu.make_async_copy(dst_hbm.at[pl.ds(ebase + i * _CHUNK, _CHUNK)],
                                 didx[g], isem_d[g])

  def gather(g, b):
    return pltpu.make_async_copy(h_hbm.at[sidx[g]], rows[b], gsem[b])

  def scatter(g, b):
    return pltpu.make_async_copy(rows[b], accum.at[didx[g]], ssem[b])

  def when(cond, fn):
    if isinstance(cond, bool):
      if cond:
        fn()
    else:
      pl.when(cond)(fn)

  def chunk_step(i, b):
    rb = b % _NBUF
    g = b
    bn = (b + 2) % _NBUF
    gn = (b + 2) % _IGEN
    gp = (b + 6) % _IGEN
    gf = (b + 4) % _IGEN
    when(i >= 2, lambda: scatter(gp, bn).wait())

    def stage_idx():
      idx_src(i + 4, gf).start()
      idx_dst(i + 4, gf).start()

    when(i + 4 < _NCH, stage_idx)

    def launch_gather():
      idx_src(i + 2, gn).wait()
      idx_dst(i + 2, gn).wait()
      gather(gn, bn).start()

    when(i + 2 < _NCH, launch_gather)
    gather(g, rb).wait()
    pltpu.async_copy(rows[rb], accum.at[didx[g]], ssem[rb], add=True)

  for j in range(4):
    idx_src(j, j).start()
    idx_dst(j, j).start()
  for j in range(2):
    idx_src(j, j).wait()
    idx_dst(j, j).wait()
    gather(j, j).start()

  def outer(k, carry):
    for b in range(_UNROLL):
      chunk_step(k * _UNROLL + b, b)
    return carry

  lax.fori_loop(0, _NFULL // _UNROLL, outer, 0)
  for i in range(_NFULL, _NCH):
    chunk_step(i, i % _IGEN)
  scatter((_NCH - 2) % _IGEN, (_NCH - 2) % _NBUF).wait()
  scatter((_NCH - 1) % _IGEN, (_NCH - 1) % _NBUF).wait()

  plsc.subcore_barrier()
  pltpu.sync_copy(accum.at[pl.ds(r0, _RPT)],
                  out_hbm.at[pl.ds(c * _N + r0, _RPT)])

  @pl.when(s == _NS - 1)
  def _():
    pltpu.sync_copy(accum.at[pl.ds(tail, _N - tail)],
                    out_hbm.at[pl.ds(c * _N + tail, _N - tail)])


_segsum = functools.partial(
    pl.kernel,
    out_type=jax.ShapeDtypeStruct((2 * _N, _H), jnp.float32),
    mesh=plsc.VectorSubcoreMesh(core_axis_name="c", subcore_axis_name="s"),
    scratch_types=[pltpu.VMEM((_CHUNK, _H), jnp.float32)] * _NBUF
    + [pltpu.VMEM((_CHUNK,), jnp.int32)] * (2 * _IGEN)
    + [pltpu.SemaphoreType.DMA] * (2 * _NBUF + 2 * _IGEN)
    + [pltpu.VMEM_SHARED((_N, _H), jnp.float32)],
)(_segsum_body)


def _bn_relu(y, g, be):
  m = jnp.mean(y, axis=0, keepdims=True)
  v = jnp.mean((y - m) * (y - m), axis=0, keepdims=True)
  return jnp.maximum(g * (y - m) * lax.rsqrt(v + 1e-5) + be, 0.0)


def _dense_body(x_ref, w1_ref, b1_ref, g1_ref, be1_ref, w2_ref, b2_ref,
                g2_ref, be2_ref, out_ref):
  x = x_ref[:_N] + x_ref[_N:]
  y = jnp.dot(x, w1_ref[...], preferred_element_type=jnp.float32) + b1_ref[...]
  y = _bn_relu(y, g1_ref[...], be1_ref[...])
  z = jnp.dot(y, w2_ref[...], preferred_element_type=jnp.float32) + b2_ref[...]
  out_ref[...] = _bn_relu(z, g2_ref[...], be2_ref[...])


def _dense(x, w1, b1, g1, be1, w2, b2, g2, be2):
  return pl.pallas_call(
      _dense_body,
      out_shape=jax.ShapeDtypeStruct((_N, _H), jnp.float32),
  )(x, w1, b1.reshape(1, _H), g1.reshape(1, _H), be1.reshape(1, _H),
    w2, b2.reshape(1, _H), g2.reshape(1, _H), be2.reshape(1, _H))


def _loss_body(h1_ref, h2_ref, msk_ref, bias1_ref, bias2_ref, lbl1_ref,
               lbl2_ref, dw_ref, db_ref, out_ref):
  h1 = h1_ref[...]
  h2 = h2_ref[...]
  msk = msk_ref[...]
  c = jnp.sum(h1 * msk, axis=0, keepdims=True) / jnp.sum(msk)
  c = 1.0 / (1.0 + jnp.exp(-c))
  cw = jnp.dot(c, dw_ref[...], preferred_element_type=jnp.float32)
  db = db_ref[0, 0]

  def bce(h, bias, lbl):
    logits = jnp.sum(h * cw, axis=1, keepdims=True) + db + bias
    return jnp.sum(jnp.maximum(logits, 0.0) - logits * lbl
                   + jnp.log(1.0 + jnp.exp(-jnp.abs(logits))))

  total = bce(h1, bias1_ref[...], lbl1_ref[...]) + bce(
      h2, bias2_ref[...], lbl2_ref[...])
  out_ref[...] = (total / (2.0 * _N)).reshape(1, 1)


def _loss(h1, h2, msk, bias1, bias2, lbl1, lbl2, dw, db):
  return pl.pallas_call(
      _loss_body,
      out_shape=jax.ShapeDtypeStruct((1, 1), jnp.float32),
  )(h1, h2, msk, bias1, bias2, lbl1, lbl2, dw, db)


def kernel(seq1, seq2, adj, msk, samp_bias1, samp_bias2, lbl,
           gin0_W1, gin0_b1, gin0_g1, gin0_be1, gin0_W2, gin0_b2, gin0_g2,
           gin0_be2, gin1_W1, gin1_b1, gin1_g1, gin1_be1, gin1_W2, gin1_b2,
           gin1_g2, gin1_be2, disc_W, disc_b):
  src, dst = adj[0], adj[1]
  zeros = jnp.zeros((_N, _H), jnp.float32)

  p0a = _segsum(seq1, zeros, src, dst)
  p0b = _segsum(seq2, zeros, src, dst)
  hA = _dense(p0a, gin0_W1, gin0_b1, gin0_g1, gin0_be1,
              gin0_W2, gin0_b2, gin0_g2, gin0_be2)
  hB = _dense(p0b, gin0_W1, gin0_b1, gin0_g1, gin0_be1,
              gin0_W2, gin0_b2, gin0_g2, gin0_be2)
  p1a = _segsum(hA, zeros, src, dst)
  p1b = _segsum(hB, zeros, src, dst)
  hA = _dense(p1a, gin1_W1, gin1_b1, gin1_g1, gin1_be1,
              gin1_W2, gin1_b2, gin1_g2, gin1_be2)
  hB = _dense(p1b, gin1_W1, gin1_b1, gin1_g1, gin1_be1,
              gin1_W2, gin1_b2, gin1_g2, gin1_be2)

  out = _loss(hA, hB, msk.reshape(_N, 1),
              samp_bias1.reshape(_N, 1), samp_bias2.reshape(_N, 1),
              lbl[0, :_N].reshape(_N, 1), lbl[0, _N:].reshape(_N, 1),
              disc_W, disc_b.reshape(1, 1))
  return out[0, 0]


# speedup vs baseline: 1.0896x; 1.0896x over previous
import functools
import jax
import jax.numpy as jnp
from jax import lax
from jax.experimental import pallas as pl
from jax.experimental.pallas import tpu as pltpu
from jax.experimental.pallas import tpu_sc as plsc

_N = 10000
_E = 320000
_H = 128
_NS = 16
_EPT = _E // _NS
_CHUNK = 80
_NCH = _EPT // _CHUNK
_RPT = 624


_NBUF = 4
_IGEN = 8
_UNROLL = 8
_NFULL = 248


def _segsum_body(h_hbm, src_hbm, dst_hbm, out_hbm, *scr):
  rows = scr[0:4]
  sidx = scr[4:12]
  didx = scr[12:20]
  gsem = scr[20:24]
  ssem = scr[24:28]
  isem_s = scr[28:36]
  isem_d = scr[36:44]
  accum = scr[44]
  c = lax.axis_index("c")
  s = lax.axis_index("s")
  r0 = s * _RPT
  tail = _NS * _RPT
  pltpu.sync_copy(h_hbm.at[pl.ds(c * _N + r0, _RPT)],
                  accum.at[pl.ds(r0, _RPT)])

  @pl.when(s == _NS - 1)
  def _():
    pltpu.sync_copy(h_hbm.at[pl.ds(c * _N + tail, _N - tail)],
                    accum.at[pl.ds(tail, _N - tail)])

  plsc.subcore_barrier()

  def idx_src(i, g):
    return pltpu.make_async_copy(src_hbm.at[c, s, i], sidx[g], isem_s[g])

  def idx_dst(i, g):
    return pltpu.make_async_copy(dst_hbm.at[s, i], didx[g], isem_d[g])

  def gather(g, b):
    return pltpu.make_async_copy(h_hbm.at[sidx[g]], rows[b], gsem[b])

  def scatter(g, b):
    return pltpu.make_async_copy(rows[b], accum.at[didx[g]], ssem[b])

  for j in range(4):
    idx_src(j, j).start()
    idx_dst(j, j).start()
  for j in range(2):
    idx_src(j, j).wait()
    idx_dst(j, j).wait()
    gather(j, j).start()

  def outer(k, carry):
    for b in range(_UNROLL):
      i = k * _UNROLL + b
      rb = b % _NBUF
      g = b
      bn = (b + 2) % _NBUF
      gn = (b + 2) % _IGEN
      gp = (b + 6) % _IGEN
      gf = (b + 4) % _IGEN

      @pl.when(i >= 2)
      def _():
        scatter(gp, bn).wait()

      @pl.when(i + 4 < _NCH)
      def _():
        idx_src(i + 4, gf).start()
        idx_dst(i + 4, gf).start()

      @pl.when(i + 2 < _NCH)
      def _():
        idx_src(i + 2, gn).wait()
        idx_dst(i + 2, gn).wait()
        gather(gn, bn).start()

      gather(g, rb).wait()
      pltpu.async_copy(rows[rb], accum.at[didx[g]], ssem[rb], add=True)
    return carry

  lax.fori_loop(0, _NFULL // _UNROLL, outer, 0)

  scatter(6, 2).wait()
  gather(0, 0).wait()
  pltpu.async_copy(rows[0], accum.at[didx[0]], ssem[0], add=True)
  scatter(7, 3).wait()
  gather(1, 1).wait()
  pltpu.async_copy(rows[1], accum.at[didx[1]], ssem[1], add=True)
  scatter(0, 0).wait()
  scatter(1, 1).wait()

  plsc.subcore_barrier()
  pltpu.sync_copy(accum.at[pl.ds(r0, _RPT)],
                  out_hbm.at[pl.ds(c * _N + r0, _RPT)])

  @pl.when(s == _NS - 1)
  def _():
    pltpu.sync_copy(accum.at[pl.ds(tail, _N - tail)],
                    out_hbm.at[pl.ds(c * _N + tail, _N - tail)])


_segsum = functools.partial(
    pl.kernel,
    out_type=jax.ShapeDtypeStruct((2 * _N, _H), jnp.float32),
    mesh=plsc.VectorSubcoreMesh(core_axis_name="c", subcore_axis_name="s"),
    scratch_types=[pltpu.VMEM((_CHUNK, _H), jnp.float32)] * _NBUF
    + [pltpu.VMEM((_CHUNK,), jnp.int32)] * (2 * _IGEN)
    + [pltpu.SemaphoreType.DMA] * (2 * _NBUF + 2 * _IGEN)
    + [pltpu.VMEM_SHARED((_N, _H), jnp.float32)],
)(_segsum_body)


def _bn_relu(y, g, be):
  m = jnp.mean(y, axis=0, keepdims=True)
  v = jnp.mean((y - m) * (y - m), axis=0, keepdims=True)
  return jnp.maximum(g * (y - m) * lax.rsqrt(v + 1e-5) + be, 0.0)


def _dense_body(x_ref, w1_ref, b1_ref, g1_ref, be1_ref, w2_ref, b2_ref,
                g2_ref, be2_ref, out_ref):
  x = x_ref[...]
  y = jnp.dot(x, w1_ref[...], preferred_element_type=jnp.float32) + b1_ref[...]
  y = _bn_relu(y, g1_ref[...], be1_ref[...])
  z = jnp.dot(y, w2_ref[...], preferred_element_type=jnp.float32) + b2_ref[...]
  out_ref[...] = _bn_relu(z, g2_ref[...], be2_ref[...])


def _dense(x, w1, b1, g1, be1, w2, b2, g2, be2):
  full = lambda i: (0, 0)
  return pl.pallas_call(
      _dense_body,
      grid=(2,),
      in_specs=[
          pl.BlockSpec((_N, _H), lambda i: (i, 0)),
          pl.BlockSpec((_H, _H), full),
          pl.BlockSpec((1, _H), full),
          pl.BlockSpec((1, _H), full),
          pl.BlockSpec((1, _H), full),
          pl.BlockSpec((_H, _H), full),
          pl.BlockSpec((1, _H), full),
          pl.BlockSpec((1, _H), full),
          pl.BlockSpec((1, _H), full),
      ],
      out_specs=pl.BlockSpec((_N, _H), lambda i: (i, 0)),
      out_shape=jax.ShapeDtypeStruct((2 * _N, _H), jnp.float32),
  )(x, w1, b1.reshape(1, _H), g1.reshape(1, _H), be1.reshape(1, _H),
    w2, b2.reshape(1, _H), g2.reshape(1, _H), be2.reshape(1, _H))


def _dense_loss_body(x_ref, w1_ref, b1_ref, g1_ref, be1_ref, w2_ref, b2_ref,
                     g2_ref, be2_ref, msk_ref, bias_ref, lbl_ref, dw_ref,
                     db_ref, out_ref, cw_ref):
  i = pl.program_id(0)
  x = x_ref[...]
  y = jnp.dot(x, w1_ref[...], preferred_element_type=jnp.float32) + b1_ref[...]
  y = _bn_relu(y, g1_ref[...], be1_ref[...])
  z = jnp.dot(y, w2_ref[...], preferred_element_type=jnp.float32) + b2_ref[...]
  h = _bn_relu(z, g2_ref[...], be2_ref[...])

  @pl.when(i == 0)
  def _():
    msk = msk_ref[...]
    c = jnp.sum(h * msk, axis=0, keepdims=True) / jnp.sum(msk)
    c = 1.0 / (1.0 + jnp.exp(-c))
    cw_ref[...] = jnp.dot(c, dw_ref[...], preferred_element_type=jnp.float32)
    out_ref[...] = jnp.zeros((1, 1), jnp.float32)

  logits = (jnp.sum(h * cw_ref[...], axis=1, keepdims=True) + db_ref[0, 0]
            + bias_ref[...])
  per = (jnp.maximum(logits, 0.0) - logits * lbl_ref[...]
         + jnp.log(1.0 + jnp.exp(-jnp.abs(logits))))
  out_ref[...] += (jnp.sum(per) / (2.0 * _N)).reshape(1, 1)


def _dense_loss(x, w1, b1, g1, be1, w2, b2, g2, be2, msk, bias, lblr, dw, db):
  full = lambda i: (0, 0)
  return pl.pallas_call(
      _dense_loss_body,
      grid=(2,),
      in_specs=[
          pl.BlockSpec((_N, _H), lambda i: (i, 0)),
          pl.BlockSpec((_H, _H), full),
          pl.BlockSpec((1, _H), full),
          pl.BlockSpec((1, _H), full),
          pl.BlockSpec((1, _H), full),
          pl.BlockSpec((_H, _H), full),
          pl.BlockSpec((1, _H), full),
          pl.BlockSpec((1, _H), full),
          pl.BlockSpec((1, _H), full),
          pl.BlockSpec((_N, 1), full),
          pl.BlockSpec((_N, 1), lambda i: (i, 0)),
          pl.BlockSpec((_N, 1), lambda i: (i, 0)),
          pl.BlockSpec((_H, _H), full),
          pl.BlockSpec((1, 1), full),
      ],
      out_specs=pl.BlockSpec((1, 1), full),
      out_shape=jax.ShapeDtypeStruct((1, 1), jnp.float32),
      scratch_shapes=[pltpu.VMEM((1, _H), jnp.float32)],
  )(x, w1, b1.reshape(1, _H), g1.reshape(1, _H), be1.reshape(1, _H),
    w2, b2.reshape(1, _H), g2.reshape(1, _H), be2.reshape(1, _H),
    msk, bias, lblr, dw, db)


def kernel(seq1, seq2, adj, msk, samp_bias1, samp_bias2, lbl,
           gin0_W1, gin0_b1, gin0_g1, gin0_be1, gin0_W2, gin0_b2, gin0_g2,
           gin0_be2, gin1_W1, gin1_b1, gin1_g1, gin1_be1, gin1_W2, gin1_b2,
           gin1_g2, gin1_be2, disc_W, disc_b):
  src, dst = adj[0], adj[1]
  src2 = jnp.concatenate([src, src + _N])
  src2 = src2.reshape(2, _NS, _NCH, _CHUNK)
  dst = dst.reshape(_NS, _NCH, _CHUNK)
  h0 = jnp.concatenate([seq1, seq2], axis=0)

  pooled0 = _segsum(h0, src2, dst)
  hA = _dense(pooled0, gin0_W1, gin0_b1, gin0_g1, gin0_be1,
              gin0_W2, gin0_b2, gin0_g2, gin0_be2)
  pooled1 = _segsum(hA, src2, dst)

  bias = jnp.concatenate([samp_bias1, samp_bias2], axis=1).reshape(2 * _N, 1)
  out = _dense_loss(pooled1, gin1_W1, gin1_b1, gin1_g1, gin1_be1,
                    gin1_W2, gin1_b2, gin1_g2, gin1_be2,
                    msk.reshape(_N, 1), bias, lbl.reshape(2 * _N, 1),
                    disc_W, disc_b.reshape(1, 1))
  return out[0, 0]


# speedup vs baseline: 1.1224x; 1.0301x over previous
import functools
import jax
import jax.numpy as jnp
from jax import lax
from jax.experimental import pallas as pl
from jax.experimental.pallas import tpu as pltpu
from jax.experimental.pallas import tpu_sc as plsc

_N = 10000
_E = 320000
_H = 128
_NS = 16
_EPT = _E // _NS
_CHUNK = 80
_NCH = _EPT // _CHUNK
_RPT = 624


_NBUF = 4
_IGEN = 8
_UNROLL = 8
_NFULL = 248


def _segsum_body(h_hbm, src_hbm, dst_hbm, out_hbm, *scr):
  rows = scr[0:4]
  sidx = scr[4:12]
  didx = scr[12:20]
  gsem = scr[20:24]
  ssem = scr[24:28]
  isem_s = scr[28:36]
  isem_d = scr[36:44]
  accum = scr[44]
  c = lax.axis_index("c")
  s = lax.axis_index("s")
  r0 = s * _RPT
  tail = _NS * _RPT
  pltpu.sync_copy(h_hbm.at[pl.ds(c * _N + r0, _RPT)],
                  accum.at[pl.ds(r0, _RPT)])

  @pl.when(s == _NS - 1)
  def _():
    pltpu.sync_copy(h_hbm.at[pl.ds(c * _N + tail, _N - tail)],
                    accum.at[pl.ds(tail, _N - tail)])

  plsc.subcore_barrier()

  def idx_src(i, g):
    return pltpu.make_async_copy(src_hbm.at[c, s, i], sidx[g], isem_s[g])

  def idx_dst(i, g):
    return pltpu.make_async_copy(dst_hbm.at[s, i], didx[g], isem_d[g])

  def gather(g, b):
    return pltpu.make_async_copy(h_hbm.at[sidx[g]], rows[b], gsem[b])

  def scatter(g, b):
    return pltpu.make_async_copy(rows[b], accum.at[didx[g]], ssem[b])

  for j in range(4):
    idx_src(j, j).start()
    idx_dst(j, j).start()
  for j in range(2):
    idx_src(j, j).wait()
    idx_dst(j, j).wait()
    gather(j, j).start()

  def outer(k, carry):
    for b in range(_UNROLL):
      i = k * _UNROLL + b
      rb = b % _NBUF
      g = b
      bn = (b + 2) % _NBUF
      gn = (b + 2) % _IGEN
      gp = (b + 6) % _IGEN
      gf = (b + 4) % _IGEN

      @pl.when(i >= 2)
      def _():
        scatter(gp, bn).wait()

      @pl.when(i + 4 < _NCH)
      def _():
        idx_src(i + 4, gf).start()
        idx_dst(i + 4, gf).start()

      @pl.when(i + 2 < _NCH)
      def _():
        idx_src(i + 2, gn).wait()
        idx_dst(i + 2, gn).wait()
        gather(gn, bn).start()

      gather(g, rb).wait()
      pltpu.async_copy(rows[rb], accum.at[didx[g]], ssem[rb], add=True)
    return carry

  lax.fori_loop(0, _NFULL // _UNROLL, outer, 0)

  scatter(6, 2).wait()
  gather(0, 0).wait()
  pltpu.async_copy(rows[0], accum.at[didx[0]], ssem[0], add=True)
  scatter(7, 3).wait()
  gather(1, 1).wait()
  pltpu.async_copy(rows[1], accum.at[didx[1]], ssem[1], add=True)
  scatter(0, 0).wait()
  scatter(1, 1).wait()

  plsc.subcore_barrier()
  pltpu.sync_copy(accum.at[pl.ds(r0, _RPT)],
                  out_hbm.at[pl.ds(c * _N + r0, _RPT)])

  @pl.when(s == _NS - 1)
  def _():
    pltpu.sync_copy(accum.at[pl.ds(tail, _N - tail)],
                    out_hbm.at[pl.ds(c * _N + tail, _N - tail)])


_segsum = functools.partial(
    pl.kernel,
    out_type=jax.ShapeDtypeStruct((2 * _N, _H), jnp.float32),
    mesh=plsc.VectorSubcoreMesh(core_axis_name="c", subcore_axis_name="s"),
    scratch_types=[pltpu.VMEM((_CHUNK, _H), jnp.float32)] * _NBUF
    + [pltpu.VMEM((_CHUNK,), jnp.int32)] * (2 * _IGEN)
    + [pltpu.SemaphoreType.DMA] * (2 * _NBUF + 2 * _IGEN)
    + [pltpu.VMEM_SHARED((_N, _H), jnp.float32)],
)(_segsum_body)


def _bn_relu(y, g, be):
  m = jnp.mean(y, axis=0, keepdims=True)
  v = jnp.mean((y - m) * (y - m), axis=0, keepdims=True)
  return jnp.maximum(g * (y - m) * lax.rsqrt(v + 1e-5) + be, 0.0)


def _dense_body(x_ref, w1_ref, b1_ref, g1_ref, be1_ref, w2_ref, b2_ref,
                g2_ref, be2_ref, out_ref):
  x = x_ref[...]
  y = jnp.dot(x, w1_ref[...], preferred_element_type=jnp.float32) + b1_ref[...]
  y = _bn_relu(y, g1_ref[...], be1_ref[...])
  z = jnp.dot(y, w2_ref[...], preferred_element_type=jnp.float32) + b2_ref[...]
  out_ref[...] = _bn_relu(z, g2_ref[...], be2_ref[...])


def _dense(x, w1, b1, g1, be1, w2, b2, g2, be2):
  full = lambda i: (0, 0)
  return pl.pallas_call(
      _dense_body,
      grid=(2,),
      in_specs=[
          pl.BlockSpec((_N, _H), lambda i: (i, 0)),
          pl.BlockSpec((_H, _H), full),
          pl.BlockSpec((1, _H), full),
          pl.BlockSpec((1, _H), full),
          pl.BlockSpec((1, _H), full),
          pl.BlockSpec((_H, _H), full),
          pl.BlockSpec((1, _H), full),
          pl.BlockSpec((1, _H), full),
          pl.BlockSpec((1, _H), full),
      ],
      out_specs=pl.BlockSpec((_N, _H), lambda i: (i, 0)),
      out_shape=jax.ShapeDtypeStruct((2 * _N, _H), jnp.float32),
  )(x, w1, b1.reshape(1, _H), g1.reshape(1, _H), be1.reshape(1, _H),
    w2, b2.reshape(1, _H), g2.reshape(1, _H), be2.reshape(1, _H))


def _dense_loss_body(x_ref, w1_ref, b1_ref, g1_ref, be1_ref, w2_ref, b2_ref,
                     g2_ref, be2_ref, msk_ref, bias_ref, lbl_ref, dw_ref,
                     db_ref, out_ref, cw_ref):
  i = pl.program_id(0)
  x = x_ref[...]
  y = jnp.dot(x, w1_ref[...], preferred_element_type=jnp.float32) + b1_ref[...]
  y = _bn_relu(y, g1_ref[...], be1_ref[...])
  z = jnp.dot(y, w2_ref[...], preferred_element_type=jnp.float32) + b2_ref[...]
  h = _bn_relu(z, g2_ref[...], be2_ref[...])

  @pl.when(i == 0)
  def _():
    c = (jnp.dot(msk_ref[...], h, preferred_element_type=jnp.float32)
         / jnp.sum(msk_ref[...]))
    c = 1.0 / (1.0 + jnp.exp(-c))
    cw_ref[...] = jnp.dot(c, dw_ref[...], preferred_element_type=jnp.float32)
    out_ref[...] = jnp.zeros((1, 1), jnp.float32)

  scores = lax.dot_general(cw_ref[...], h, (((1,), (1,)), ((), ())),
                           preferred_element_type=jnp.float32)
  logits = scores + db_ref[0, 0] + bias_ref[0]
  per = (jnp.maximum(logits, 0.0) - logits * lbl_ref[0]
         + jnp.log(1.0 + jnp.exp(-jnp.abs(logits))))
  out_ref[...] += (jnp.sum(per) / (2.0 * _N)).reshape(1, 1)


def _dense_loss(x, w1, b1, g1, be1, w2, b2, g2, be2, msk, bias, lblr, dw, db):
  full = lambda i: (0, 0)
  return pl.pallas_call(
      _dense_loss_body,
      grid=(2,),
      in_specs=[
          pl.BlockSpec((_N, _H), lambda i: (i, 0)),
          pl.BlockSpec((_H, _H), full),
          pl.BlockSpec((1, _H), full),
          pl.BlockSpec((1, _H), full),
          pl.BlockSpec((1, _H), full),
          pl.BlockSpec((_H, _H), full),
          pl.BlockSpec((1, _H), full),
          pl.BlockSpec((1, _H), full),
          pl.BlockSpec((1, _H), full),
          pl.BlockSpec((1, _N), full),
          pl.BlockSpec((1, 1, _N), lambda i: (i, 0, 0)),
          pl.BlockSpec((1, 1, _N), lambda i: (i, 0, 0)),
          pl.BlockSpec((_H, _H), full),
          pl.BlockSpec((1, 1), full),
      ],
      out_specs=pl.BlockSpec((1, 1), full),
      out_shape=jax.ShapeDtypeStruct((1, 1), jnp.float32),
      scratch_shapes=[pltpu.VMEM((1, _H), jnp.float32)],
  )(x, w1, b1.reshape(1, _H), g1.reshape(1, _H), be1.reshape(1, _H),
    w2, b2.reshape(1, _H), g2.reshape(1, _H), be2.reshape(1, _H),
    msk, bias, lblr, dw, db)


def kernel(seq1, seq2, adj, msk, samp_bias1, samp_bias2, lbl,
           gin0_W1, gin0_b1, gin0_g1, gin0_be1, gin0_W2, gin0_b2, gin0_g2,
           gin0_be2, gin1_W1, gin1_b1, gin1_g1, gin1_be1, gin1_W2, gin1_b2,
           gin1_g2, gin1_be2, disc_W, disc_b):
  src, dst = adj[0], adj[1]
  src2 = jnp.concatenate([src, src + _N])
  src2 = src2.reshape(2, _NS, _NCH, _CHUNK)
  dst = dst.reshape(_NS, _NCH, _CHUNK)
  h0 = jnp.concatenate([seq1, seq2], axis=0)

  pooled0 = _segsum(h0, src2, dst)
  hA = _dense(pooled0, gin0_W1, gin0_b1, gin0_g1, gin0_be1,
              gin0_W2, gin0_b2, gin0_g2, gin0_be2)
  pooled1 = _segsum(hA, src2, dst)

  bias = jnp.stack([samp_bias1, samp_bias2])
  out = _dense_loss(pooled1, gin1_W1, gin1_b1, gin1_g1, gin1_be1,
                    gin1_W2, gin1_b2, gin1_g2, gin1_be2,
                    msk, bias, lbl.reshape(2, 1, _N), disc_W,
                    disc_b.reshape(1, 1))
  return out[0, 0]


# speedup vs baseline: 1.1433x; 1.0186x over previous
import functools
import jax
import jax.numpy as jnp
from jax import lax
from jax.experimental import pallas as pl
from jax.experimental.pallas import tpu as pltpu
from jax.experimental.pallas import tpu_sc as plsc

_N = 10000
_E = 320000
_H = 128
_NS = 16
_EPT = _E // _NS
_CHUNK = 80
_NCH = _EPT // _CHUNK
_RPT = 624


_NBUF = 4
_IGEN = 8
_UNROLL = 8
_NFULL = 248


def _segsum_body(h_hbm, src_hbm, dst_hbm, out_hbm, *scr):
  rows = scr[0:4]
  sidx = scr[4:12]
  didx = scr[12:20]
  gsem = scr[20:24]
  ssem = scr[24:28]
  isem_s = scr[28:36]
  isem_d = scr[36:44]
  accum = scr[44]
  c = lax.axis_index("c")
  s = lax.axis_index("s")
  r0 = s * _RPT
  tail = _NS * _RPT
  pltpu.sync_copy(h_hbm.at[pl.ds(c * _N + r0, _RPT)],
                  accum.at[pl.ds(r0, _RPT)])

  @pl.when(s == _NS - 1)
  def _():
    pltpu.sync_copy(h_hbm.at[pl.ds(c * _N + tail, _N - tail)],
                    accum.at[pl.ds(tail, _N - tail)])

  plsc.subcore_barrier()

  def idx_src(i, g):
    return pltpu.make_async_copy(src_hbm.at[c, s, i], sidx[g], isem_s[g])

  def idx_dst(i, g):
    return pltpu.make_async_copy(dst_hbm.at[s, i], didx[g], isem_d[g])

  def gather(g, b):
    return pltpu.make_async_copy(h_hbm.at[sidx[g]], rows[b], gsem[b])

  def scatter(g, b):
    return pltpu.make_async_copy(rows[b], accum.at[didx[g]], ssem[b])

  for j in range(4):
    idx_src(j, j).start()
    idx_dst(j, j).start()
  for j in range(2):
    idx_src(j, j).wait()
    idx_dst(j, j).wait()
    gather(j, j).start()

  def outer(k, carry):
    for b in range(_UNROLL):
      i = k * _UNROLL + b
      rb = b % _NBUF
      g = b
      bn = (b + 2) % _NBUF
      gn = (b + 2) % _IGEN
      gp = (b + 6) % _IGEN
      gf = (b + 4) % _IGEN

      @pl.when(i >= 2)
      def _():
        scatter(gp, bn).wait()

      @pl.when(i + 4 < _NCH)
      def _():
        idx_src(i + 4, gf).start()
        idx_dst(i + 4, gf).start()

      @pl.when(i + 2 < _NCH)
      def _():
        idx_src(i + 2, gn).wait()
        idx_dst(i + 2, gn).wait()
        gather(gn, bn).start()

      gather(g, rb).wait()
      pltpu.async_copy(rows[rb], accum.at[didx[g]], ssem[rb], add=True)
    return carry

  lax.fori_loop(0, _NFULL // _UNROLL, outer, 0)

  scatter(6, 2).wait()
  gather(0, 0).wait()
  pltpu.async_copy(rows[0], accum.at[didx[0]], ssem[0], add=True)
  scatter(7, 3).wait()
  gather(1, 1).wait()
  pltpu.async_copy(rows[1], accum.at[didx[1]], ssem[1], add=True)
  scatter(0, 0).wait()
  scatter(1, 1).wait()

  plsc.subcore_barrier()
  pltpu.sync_copy(accum.at[pl.ds(r0, _RPT)],
                  out_hbm.at[pl.ds(c * _N + r0, _RPT)])

  @pl.when(s == _NS - 1)
  def _():
    pltpu.sync_copy(accum.at[pl.ds(tail, _N - tail)],
                    out_hbm.at[pl.ds(c * _N + tail, _N - tail)])


_segsum = functools.partial(
    pl.kernel,
    out_type=jax.ShapeDtypeStruct((2 * _N, _H), jnp.float32),
    mesh=plsc.VectorSubcoreMesh(core_axis_name="c", subcore_axis_name="s"),
    scratch_types=[pltpu.VMEM((_CHUNK, _H), jnp.float32)] * _NBUF
    + [pltpu.VMEM((_CHUNK,), jnp.int32)] * (2 * _IGEN)
    + [pltpu.SemaphoreType.DMA] * (2 * _NBUF + 2 * _IGEN)
    + [pltpu.VMEM_SHARED((_N, _H), jnp.float32)],
)(_segsum_body)


def _bn_relu(y, g, be):
  m = jnp.mean(y, axis=0, keepdims=True)
  v = jnp.mean(y * y, axis=0, keepdims=True) - m * m
  return jnp.maximum(g * (y - m) * lax.rsqrt(v + 1e-5) + be, 0.0)


def _dense_body(x_ref, w1_ref, b1_ref, g1_ref, be1_ref, w2_ref, b2_ref,
                g2_ref, be2_ref, out_ref):
  x = x_ref[...]
  y = jnp.dot(x, w1_ref[...], preferred_element_type=jnp.float32) + b1_ref[...]
  y = _bn_relu(y, g1_ref[...], be1_ref[...])
  z = jnp.dot(y, w2_ref[...], preferred_element_type=jnp.float32) + b2_ref[...]
  out_ref[...] = _bn_relu(z, g2_ref[...], be2_ref[...])


def _dense(x, w1, b1, g1, be1, w2, b2, g2, be2):
  full = lambda i: (0, 0)
  return pl.pallas_call(
      _dense_body,
      grid=(2,),
      in_specs=[
          pl.BlockSpec((_N, _H), lambda i: (i, 0)),
          pl.BlockSpec((_H, _H), full),
          pl.BlockSpec((1, _H), full),
          pl.BlockSpec((1, _H), full),
          pl.BlockSpec((1, _H), full),
          pl.BlockSpec((_H, _H), full),
          pl.BlockSpec((1, _H), full),
          pl.BlockSpec((1, _H), full),
          pl.BlockSpec((1, _H), full),
      ],
      out_specs=pl.BlockSpec((_N, _H), lambda i: (i, 0)),
      out_shape=jax.ShapeDtypeStruct((2 * _N, _H), jnp.float32),
  )(x, w1, b1.reshape(1, _H), g1.reshape(1, _H), be1.reshape(1, _H),
    w2, b2.reshape(1, _H), g2.reshape(1, _H), be2.reshape(1, _H))


def _dense_loss_body(x_ref, w1_ref, b1_ref, g1_ref, be1_ref, w2_ref, b2_ref,
                     g2_ref, be2_ref, msk_ref, bias_ref, lbl_ref, dw_ref,
                     db_ref, out_ref, cw_ref):
  i = pl.program_id(0)
  x = x_ref[...]
  y = jnp.dot(x, w1_ref[...], preferred_element_type=jnp.float32) + b1_ref[...]
  y = _bn_relu(y, g1_ref[...], be1_ref[...])
  z = jnp.dot(y, w2_ref[...], preferred_element_type=jnp.float32) + b2_ref[...]
  h = _bn_relu(z, g2_ref[...], be2_ref[...])

  @pl.when(i == 0)
  def _():
    c = (jnp.dot(msk_ref[...], h, preferred_element_type=jnp.float32)
         / jnp.sum(msk_ref[...]))
    c = 1.0 / (1.0 + jnp.exp(-c))
    cw_ref[...] = jnp.dot(c, dw_ref[...], preferred_element_type=jnp.float32)
    out_ref[...] = jnp.zeros((1, 1), jnp.float32)

  scores = lax.dot_general(cw_ref[...], h, (((1,), (1,)), ((), ())),
                           preferred_element_type=jnp.float32)
  logits = scores + db_ref[0, 0] + bias_ref[0]
  per = (jnp.maximum(logits, 0.0) - logits * lbl_ref[0]
         + jnp.log(1.0 + jnp.exp(-jnp.abs(logits))))
  out_ref[...] += (jnp.sum(per) / (2.0 * _N)).reshape(1, 1)


def _dense_loss(x, w1, b1, g1, be1, w2, b2, g2, be2, msk, bias, lblr, dw, db):
  full = lambda i: (0, 0)
  return pl.pallas_call(
      _dense_loss_body,
      grid=(2,),
      in_specs=[
          pl.BlockSpec((_N, _H), lambda i: (i, 0)),
          pl.BlockSpec((_H, _H), full),
          pl.BlockSpec((1, _H), full),
          pl.BlockSpec((1, _H), full),
          pl.BlockSpec((1, _H), full),
          pl.BlockSpec((_H, _H), full),
          pl.BlockSpec((1, _H), full),
          pl.BlockSpec((1, _H), full),
          pl.BlockSpec((1, _H), full),
          pl.BlockSpec((1, _N), full),
          pl.BlockSpec((1, 1, _N), lambda i: (i, 0, 0)),
          pl.BlockSpec((1, 1, _N), lambda i: (i, 0, 0)),
          pl.BlockSpec((_H, _H), full),
          pl.BlockSpec((1, 1), full),
      ],
      out_specs=pl.BlockSpec((1, 1), full),
      out_shape=jax.ShapeDtypeStruct((1, 1), jnp.float32),
      scratch_shapes=[pltpu.VMEM((1, _H), jnp.float32)],
  )(x, w1, b1.reshape(1, _H), g1.reshape(1, _H), be1.reshape(1, _H),
    w2, b2.reshape(1, _H), g2.reshape(1, _H), be2.reshape(1, _H),
    msk, bias, lblr, dw, db)


def kernel(seq1, seq2, adj, msk, samp_bias1, samp_bias2, lbl,
           gin0_W1, gin0_b1, gin0_g1, gin0_be1, gin0_W2, gin0_b2, gin0_g2,
           gin0_be2, gin1_W1, gin1_b1, gin1_g1, gin1_be1, gin1_W2, gin1_b2,
           gin1_g2, gin1_be2, disc_W, disc_b):
  src, dst = adj[0], adj[1]
  src2 = jnp.concatenate([src, src + _N])
  src2 = src2.reshape(2, _NS, _NCH, _CHUNK)
  dst = dst.reshape(_NS, _NCH, _CHUNK)
  h0 = jnp.concatenate([seq1, seq2], axis=0)

  pooled0 = _segsum(h0, src2, dst)
  hA = _dense(pooled0, gin0_W1, gin0_b1, gin0_g1, gin0_be1,
              gin0_W2, gin0_b2, gin0_g2, gin0_be2)
  pooled1 = _segsum(hA, src2, dst)

  bias = jnp.stack([samp_bias1, samp_bias2])
  out = _dense_loss(pooled1, gin1_W1, gin1_b1, gin1_g1, gin1_be1,
                    gin1_W2, gin1_b2, gin1_g2, gin1_be2,
                    msk, bias, lbl.reshape(2, 1, _N), disc_W,
                    disc_b.reshape(1, 1))
  return out[0, 0]


# speedup vs baseline: 1.1977x; 1.0476x over previous
import functools
import jax
import jax.numpy as jnp
from jax import lax
from jax.experimental import pallas as pl
from jax.experimental.pallas import tpu as pltpu
from jax.experimental.pallas import tpu_sc as plsc

_N = 10000
_E = 320000
_H = 128
_NS = 16
_EPT = _E // _NS
_CHUNK = 80
_NCH = _EPT // _CHUNK
_RPT = 624


_NBUF = 4
_IGEN = 8
_UNROLL = 8
_NFULL = 248


def _segsum_body(ha_hbm, hb_hbm, adj_hbm, out_hbm, *scr):
  rows = scr[0:4]
  sidx = scr[4:12]
  didx = scr[12:20]
  gsem = scr[20:24]
  ssem = scr[24:28]
  isem_s = scr[28:36]
  isem_d = scr[36:44]
  accum = scr[44]
  c = lax.axis_index("c")
  s = lax.axis_index("s")
  r0 = s * _RPT
  tail = _NS * _RPT
  sbase = s * _EPT
  dbase = _E + s * _EPT

  def idx_src(i, g):
    return pltpu.make_async_copy(
        adj_hbm.at[pl.ds(sbase + i * _CHUNK, _CHUNK)], sidx[g], isem_s[g])

  def idx_dst(i, g):
    return pltpu.make_async_copy(
        adj_hbm.at[pl.ds(dbase + i * _CHUNK, _CHUNK)], didx[g], isem_d[g])

  def scatter(g, b):
    return pltpu.make_async_copy(rows[b], accum.at[didx[g]], ssem[b])

  def run(h_hbm):
    pltpu.sync_copy(h_hbm.at[pl.ds(r0, _RPT)], accum.at[pl.ds(r0, _RPT)])

    @pl.when(s == _NS - 1)
    def _():
      pltpu.sync_copy(h_hbm.at[pl.ds(tail, _N - tail)],
                      accum.at[pl.ds(tail, _N - tail)])

    plsc.subcore_barrier()

    def gather(g, b):
      return pltpu.make_async_copy(h_hbm.at[sidx[g]], rows[b], gsem[b])

    for j in range(4):
      idx_src(j, j).start()
      idx_dst(j, j).start()
    for j in range(2):
      idx_src(j, j).wait()
      idx_dst(j, j).wait()
      gather(j, j).start()

    def outer(k, carry):
      for b in range(_UNROLL):
        i = k * _UNROLL + b
        rb = b % _NBUF
        g = b
        bn = (b + 2) % _NBUF
        gn = (b + 2) % _IGEN
        gp = (b + 6) % _IGEN
        gf = (b + 4) % _IGEN

        @pl.when(i >= 2)
        def _():
          scatter(gp, bn).wait()

        @pl.when(i + 4 < _NCH)
        def _():
          idx_src(i + 4, gf).start()
          idx_dst(i + 4, gf).start()

        @pl.when(i + 2 < _NCH)
        def _():
          idx_src(i + 2, gn).wait()
          idx_dst(i + 2, gn).wait()
          gather(gn, bn).start()

        gather(g, rb).wait()
        pltpu.async_copy(rows[rb], accum.at[didx[g]], ssem[rb], add=True)
      return carry

    lax.fori_loop(0, _NFULL // _UNROLL, outer, 0)

    scatter(6, 2).wait()
    gather(0, 0).wait()
    pltpu.async_copy(rows[0], accum.at[didx[0]], ssem[0], add=True)
    scatter(7, 3).wait()
    gather(1, 1).wait()
    pltpu.async_copy(rows[1], accum.at[didx[1]], ssem[1], add=True)
    scatter(0, 0).wait()
    scatter(1, 1).wait()

  @pl.when(c == 0)
  def _():
    run(ha_hbm)

  @pl.when(c == 1)
  def _():
    run(hb_hbm)

  plsc.subcore_barrier()
  pltpu.sync_copy(accum.at[pl.ds(r0, _RPT)],
                  out_hbm.at[pl.ds(c * _N + r0, _RPT)])

  @pl.when(s == _NS - 1)
  def _():
    pltpu.sync_copy(accum.at[pl.ds(tail, _N - tail)],
                    out_hbm.at[pl.ds(c * _N + tail, _N - tail)])


_segsum = functools.partial(
    pl.kernel,
    out_type=jax.ShapeDtypeStruct((2 * _N, _H), jnp.float32),
    mesh=plsc.VectorSubcoreMesh(core_axis_name="c", subcore_axis_name="s"),
    scratch_types=[pltpu.VMEM((_CHUNK, _H), jnp.float32)] * _NBUF
    + [pltpu.VMEM((_CHUNK,), jnp.int32)] * (2 * _IGEN)
    + [pltpu.SemaphoreType.DMA] * (2 * _NBUF + 2 * _IGEN)
    + [pltpu.VMEM_SHARED((_N, _H), jnp.float32)],
)(_segsum_body)


def _bn_relu(y, g, be):
  m = jnp.mean(y, axis=0, keepdims=True)
  v = jnp.mean(y * y, axis=0, keepdims=True) - m * m
  return jnp.maximum(g * (y - m) * lax.rsqrt(v + 1e-5) + be, 0.0)


def _dense_body(x_ref, w1_ref, b1_ref, g1_ref, be1_ref, w2_ref, b2_ref,
                g2_ref, be2_ref, out_a_ref, out_b_ref):
  i = pl.program_id(0)
  x = x_ref[...]
  y = jnp.dot(x, w1_ref[...], preferred_element_type=jnp.float32) + b1_ref[...]
  y = _bn_relu(y, g1_ref[...], be1_ref[...])
  z = jnp.dot(y, w2_ref[...], preferred_element_type=jnp.float32) + b2_ref[...]
  res = _bn_relu(z, g2_ref[...], be2_ref[...])

  @pl.when(i == 0)
  def _():
    out_a_ref[...] = res

  @pl.when(i == 1)
  def _():
    out_b_ref[...] = res


def _dense(x, w1, b1, g1, be1, w2, b2, g2, be2):
  full = lambda i: (0, 0)
  return pl.pallas_call(
      _dense_body,
      grid=(2,),
      in_specs=[
          pl.BlockSpec((_N, _H), lambda i: (i, 0)),
          pl.BlockSpec((_H, _H), full),
          pl.BlockSpec((1, _H), full),
          pl.BlockSpec((1, _H), full),
          pl.BlockSpec((1, _H), full),
          pl.BlockSpec((_H, _H), full),
          pl.BlockSpec((1, _H), full),
          pl.BlockSpec((1, _H), full),
          pl.BlockSpec((1, _H), full),
      ],
      out_specs=[pl.BlockSpec((_N, _H), full), pl.BlockSpec((_N, _H), full)],
      out_shape=[jax.ShapeDtypeStruct((_N, _H), jnp.float32),
                 jax.ShapeDtypeStruct((_N, _H), jnp.float32)],
  )(x, w1, b1.reshape(1, _H), g1.reshape(1, _H), be1.reshape(1, _H),
    w2, b2.reshape(1, _H), g2.reshape(1, _H), be2.reshape(1, _H))


def _dense_loss_body(x_ref, w1_ref, b1_ref, g1_ref, be1_ref, w2_ref, b2_ref,
                     g2_ref, be2_ref, msk_ref, bias_ref, lbl_ref, dw_ref,
                     db_ref, out_ref, cw_ref):
  i = pl.program_id(0)
  x = x_ref[...]
  y = jnp.dot(x, w1_ref[...], preferred_element_type=jnp.float32) + b1_ref[...]
  y = _bn_relu(y, g1_ref[...], be1_ref[...])
  z = jnp.dot(y, w2_ref[...], preferred_element_type=jnp.float32) + b2_ref[...]
  h = _bn_relu(z, g2_ref[...], be2_ref[...])

  @pl.when(i == 0)
  def _():
    c = (jnp.dot(msk_ref[...], h, preferred_element_type=jnp.float32)
         / jnp.sum(msk_ref[...]))
    c = 1.0 / (1.0 + jnp.exp(-c))
    cw_ref[...] = jnp.dot(c, dw_ref[...], preferred_element_type=jnp.float32)
    out_ref[...] = jnp.zeros((1, 1), jnp.float32)

  scores = lax.dot_general(cw_ref[...], h, (((1,), (1,)), ((), ())),
                           preferred_element_type=jnp.float32)
  logits = scores + db_ref[0, 0] + bias_ref[0]
  per = (jnp.maximum(logits, 0.0) - logits * lbl_ref[0]
         + jnp.log(1.0 + jnp.exp(-jnp.abs(logits))))
  out_ref[...] += (jnp.sum(per) / (2.0 * _N)).reshape(1, 1)


def _dense_loss(x, w1, b1, g1, be1, w2, b2, g2, be2, msk, bias, lblr, dw, db):
  full = lambda i: (0, 0)
  return pl.pallas_call(
      _dense_loss_body,
      grid=(2,),
      in_specs=[
          pl.BlockSpec((_N, _H), lambda i: (i, 0)),
          pl.BlockSpec((_H, _H), full),
          pl.BlockSpec((1, _H), full),
          pl.BlockSpec((1, _H), full),
          pl.BlockSpec((1, _H), full),
          pl.BlockSpec((_H, _H), full),
          pl.BlockSpec((1, _H), full),
          pl.BlockSpec((1, _H), full),
          pl.BlockSpec((1, _H), full),
          pl.BlockSpec((1, _N), full),
          pl.BlockSpec((1, 1, _N), lambda i: (i, 0, 0)),
          pl.BlockSpec((1, 1, _N), lambda i: (i, 0, 0)),
          pl.BlockSpec((_H, _H), full),
          pl.BlockSpec((1, 1), full),
      ],
      out_specs=pl.BlockSpec((1, 1), full),
      out_shape=jax.ShapeDtypeStruct((1, 1), jnp.float32),
      scratch_shapes=[pltpu.VMEM((1, _H), jnp.float32)],
  )(x, w1, b1.reshape(1, _H), g1.reshape(1, _H), be1.reshape(1, _H),
    w2, b2.reshape(1, _H), g2.reshape(1, _H), be2.reshape(1, _H),
    msk, bias, lblr, dw, db)


def kernel(seq1, seq2, adj, msk, samp_bias1, samp_bias2, lbl,
           gin0_W1, gin0_b1, gin0_g1, gin0_be1, gin0_W2, gin0_b2, gin0_g2,
           gin0_be2, gin1_W1, gin1_b1, gin1_g1, gin1_be1, gin1_W2, gin1_b2,
           gin1_g2, gin1_be2, disc_W, disc_b):
  adjf = adj.reshape(2 * _E)

  pooled0 = _segsum(seq1, seq2, adjf)
  hA_a, hA_b = _dense(pooled0, gin0_W1, gin0_b1, gin0_g1, gin0_be1,
                      gin0_W2, gin0_b2, gin0_g2, gin0_be2)
  pooled1 = _segsum(hA_a, hA_b, adjf)

  bias = jnp.stack([samp_bias1, samp_bias2])
  out = _dense_loss(pooled1, gin1_W1, gin1_b1, gin1_g1, gin1_be1,
                    gin1_W2, gin1_b2, gin1_g2, gin1_be2,
                    msk, bias, lbl.reshape(2, 1, _N), disc_W,
                    disc_b.reshape(1, 1))
  return out[0, 0]
